# baseline (speedup 1.0000x reference)
import functools

import jax
import jax.numpy as jnp
from jax import lax
from jax.experimental import pallas as pl
from jax.experimental.pallas import tpu as pltpu
from jax.experimental.pallas import tpu_sc as plsc

N, E, F, H, ED, O = 10000, 320000, 128, 256, 16, 3
NQ = 4
QW = H // NQ
NTEC = 16
EDGE_BATCH = 128
NB = 160
E_CHUNK = NB * EDGE_BATCH
E_PAD = E_CHUNK * NTEC
EA_REAL = E // 2
ACC_SLICE = 640
N_OUT = ACC_SLICE * NTEC
ROW_BLK = 400
N_BLKS = N // ROW_BLK


def _silu(v):
    return v * jax.nn.sigmoid(v)


@functools.cache
def _make_sc_round(with_edge: bool):
    mesh = plsc.VectorSubcoreMesh(core_axis_name="c", subcore_axis_name="s",
                                  num_cores=2, num_subcores=NTEC)

    out_type = [jax.ShapeDtypeStruct((N_OUT, QW), jnp.float32)
                for _ in range(NQ)]
    scratch = [
        pltpu.VMEM((NB, EDGE_BATCH), jnp.int32),
        pltpu.VMEM((NB, EDGE_BATCH), jnp.int32),
        pltpu.VMEM((EDGE_BATCH, QW), jnp.float32),
        pltpu.VMEM((EDGE_BATCH, QW), jnp.float32),
        pltpu.VMEM((EDGE_BATCH, 2 * QW), jnp.float32),
        pltpu.VMEM((EDGE_BATCH, 2 * QW), jnp.float32),
        pltpu.VMEM_SHARED((N_OUT, QW), jnp.float32),
        pltpu.SemaphoreType.DMA,
        pltpu.SemaphoreType.DMA,
        pltpu.SemaphoreType.DMA,
        pltpu.SemaphoreType.DMA,
    ]

    def body(h0, h1, h2, h3, src3, dst3, zeros, e0, e1, e2, e3,
             a0, a1, a2, a3,
             sidx, didx, row0, row1, eav0, eav1, acc,
             gs0, gs1, es0, es1):
        cid = lax.axis_index("c")
        sid = lax.axis_index("s")
        pbase = sid * (E_CHUNK // 2)
        rbase = sid * ACC_SLICE

        rows = (row0, row1)
        eavs = (eav0, eav1)
        gsem = (gs0, gs1)
        esem = (es0, es1)

        pltpu.sync_copy(src3.at[sid], sidx)
        pltpu.sync_copy(dst3.at[sid], didx)

        def gather_start(h_ref, j, b):
            pltpu.async_copy(h_ref.at[sidx.at[j]], rows[b], gsem[b])

        def gather_wait(h_ref, j, b):
            pltpu.make_async_copy(h_ref.at[sidx.at[j]], rows[b],
                                  gsem[b]).wait()

        def ea_slice(ea_ref, p):
            off = jnp.minimum(pbase + p * EDGE_BATCH, EA_REAL - EDGE_BATCH)
            return ea_ref.at[pl.ds(off, EDGE_BATCH)]

        def ea_start(ea_ref, p, eb):
            pltpu.async_copy(ea_slice(ea_ref, p), eavs[eb], esem[eb])

        def ea_wait(ea_ref, p, eb):
            pltpu.make_async_copy(ea_slice(ea_ref, p), eavs[eb],
                                  esem[eb]).wait()

        def process(j, b, eb, half):
            if with_edge:
                rb, ebuf = rows[b], eavs[eb]

                def relu_row(r, carry):
                    for c in range(QW // 16):
                        sl = pl.ds(c * 16, 16)
                        el = pl.ds(half * QW + c * 16, 16)
                        rb[r, sl] = jnp.maximum(rb[r, sl] + ebuf[r, el], 0.0)
                    return carry
                lax.fori_loop(0, EDGE_BATCH, relu_row, 0)
            pltpu.sync_copy(rows[b], acc.at[didx.at[j]], add=True)

        def run_quarter(h_ref, ea_ref, agg_ref):
            pltpu.sync_copy(zeros.at[pl.ds(rbase, ACC_SLICE)],
                            acc.at[pl.ds(rbase, ACC_SLICE)])
            plsc.subcore_barrier()

            gather_start(h_ref, 0, 0)
            if with_edge:
                ea_start(ea_ref, 0, 0)

                def step4(k, carry):
                    j = 4 * k
                    p = 2 * k
                    gather_start(h_ref, j + 1, 1)
                    ea_wait(ea_ref, p, 0)
                    gather_wait(h_ref, j, 0)
                    process(j, 0, 0, 0)
                    gather_start(h_ref, j + 2, 0)
                    ea_start(ea_ref, p + 1, 1)
                    gather_wait(h_ref, j + 1, 1)
                    process(j + 1, 1, 0, 1)
                    gather_start(h_ref, j + 3, 1)
                    ea_wait(ea_ref, p + 1, 1)
                    gather_wait(h_ref, j + 2, 0)
                    process(j + 2, 0, 1, 0)

                    @pl.when(k < NB // 4 - 1)
                    def _():
                        gather_start(h_ref, j + 4, 0)
                        ea_start(ea_ref, p + 2, 0)
                    gather_wait(h_ref, j + 3, 1)
                    process(j + 3, 1, 1, 1)
                    return carry
                lax.fori_loop(0, NB // 4, step4, 0)
            else:
                def step2(i, carry):
                    j0 = 2 * i
                    gather_start(h_ref, j0 + 1, 1)
                    gather_wait(h_ref, j0, 0)
                    process(j0, 0, 0, 0)

                    @pl.when(i < NB // 2 - 1)
                    def _():
                        gather_start(h_ref, j0 + 2, 0)
                    gather_wait(h_ref, j0 + 1, 1)
                    process(j0 + 1, 1, 0, 0)
                    return carry
                lax.fori_loop(0, NB // 2, step2, 0)
            plsc.subcore_barrier()
            pltpu.sync_copy(acc.at[pl.ds(rbase, ACC_SLICE)],
                            agg_ref.at[pl.ds(rbase, ACC_SLICE)])
            plsc.subcore_barrier()

        @pl.when(cid == 0)
        def _():
            run_quarter(h0, e0, a0)
            run_quarter(h1, e1, a1)

        @pl.when(cid == 1)
        def _():
            run_quarter(h2, e2, a2)
            run_quarter(h3, e3, a3)

    return pl.kernel(
        body, out_type=out_type, mesh=mesh, scratch_types=scratch,
        compiler_params=pltpu.CompilerParams(use_tc_tiling_on_sc=False))


def _lin1_body(x, W, b, *hq):
    h = _silu(_silu(jnp.dot(x[...], W[...],
                            preferred_element_type=jnp.float32) + b[...]))
    for q in range(NQ):
        hq[q][...] = h[:, q * QW:(q + 1) * QW]


def _lin1(x, W, b):
    return pl.pallas_call(
        _lin1_body,
        grid=(N_BLKS,),
        in_specs=[
            pl.BlockSpec((ROW_BLK, F), lambda i: (i, 0)),
            pl.BlockSpec((F, H), lambda i: (0, 0)),
            pl.BlockSpec((1, H), lambda i: (0, 0)),
        ],
        out_specs=[pl.BlockSpec((ROW_BLK, QW), lambda i: (i, 0))] * NQ,
        out_shape=[jax.ShapeDtypeStruct((N, QW), jnp.float32)] * NQ,
    )(x, W, b)


_EA_BLK = 256
_EA_BLKS = E // _EA_BLK


def _ea_body(xe, W, b, *eo):
    v = jnp.dot(xe[...], W[...], preferred_element_type=jnp.float32) + b[...]
    for q in range(NQ):
        qc = v[:, q * QW:(q + 1) * QW]
        eo[q][...] = jnp.concatenate(
            [qc[:_EA_BLK // 2, :], qc[_EA_BLK // 2:, :]], axis=1)


def _edge_lin(edge_attr, W, b):
    return pl.pallas_call(
        _ea_body,
        grid=(_EA_BLKS,),
        in_specs=[
            pl.BlockSpec((_EA_BLK, ED), lambda i: (i, 0)),
            pl.BlockSpec((ED, H), lambda i: (0, 0)),
            pl.BlockSpec((1, H), lambda i: (0, 0)),
        ],
        out_specs=[pl.BlockSpec((_EA_BLK // 2, 2 * QW), lambda i: (i, 0))] * NQ,
        out_shape=[jax.ShapeDtypeStruct((EA_REAL, 2 * QW), jnp.float32)] * NQ,
    )(edge_attr, W, b)


def _conv_body(*refs):
    hq = refs[0:NQ]
    aq = refs[NQ:2 * NQ]
    epsr, W, b = refs[2 * NQ:2 * NQ + 3]
    oq = refs[2 * NQ + 3:]
    acc = None
    for q in range(NQ):
        z = epsr[...] * hq[q][...] + aq[q][...]
        p = jnp.dot(z, W[q * QW:(q + 1) * QW, :],
                    preferred_element_type=jnp.float32)
        acc = p if acc is None else acc + p
    h = _silu(acc + b[...])
    for q in range(NQ):
        oq[q][...] = h[:, q * QW:(q + 1) * QW]


def _conv(hq, aggq, eps_row, W, b):
    return pl.pallas_call(
        _conv_body,
        grid=(N_BLKS,),
        in_specs=(
            [pl.BlockSpec((ROW_BLK, QW), lambda i: (i, 0))] * NQ
            + [pl.BlockSpec((ROW_BLK, QW), lambda i: (i, 0))] * NQ
            + [
                pl.BlockSpec((1, QW), lambda i: (0, 0)),
                pl.BlockSpec((H, H), lambda i: (0, 0)),
                pl.BlockSpec((1, H), lambda i: (0, 0)),
            ]
        ),
        out_specs=[pl.BlockSpec((ROW_BLK, QW), lambda i: (i, 0))] * NQ,
        out_shape=[jax.ShapeDtypeStruct((N, QW), jnp.float32)] * NQ,
    )(*hq, *aggq, eps_row, W, b)


def _pool_body(h0, h1, h2, h3, W1, b1, W2, b2, out, acc):
    i = pl.program_id(0)

    @pl.when(i == 0)
    def _():
        acc[...] = jnp.zeros_like(acc)

    blk = jnp.concatenate([h0[...], h1[...], h2[...], h3[...]], axis=1)
    acc[...] += jnp.sum(blk, axis=0, keepdims=True)

    @pl.when(i == N_BLKS - 1)
    def _():
        g = _silu(acc[...])
        p = _silu(jnp.dot(g, W1[...], preferred_element_type=jnp.float32)
                  + b1[...])
        out[...] = jnp.dot(p, W2[...], preferred_element_type=jnp.float32) \
            + b2[...]


def _pool_mlp(hq, W1, b1, W2p, b2p):
    return pl.pallas_call(
        _pool_body,
        grid=(N_BLKS,),
        in_specs=[pl.BlockSpec((ROW_BLK, QW), lambda i: (i, 0))] * NQ + [
            pl.BlockSpec((H, H // 2), lambda i: (0, 0)),
            pl.BlockSpec((1, H // 2), lambda i: (0, 0)),
            pl.BlockSpec((H // 2, 128), lambda i: (0, 0)),
            pl.BlockSpec((1, 128), lambda i: (0, 0)),
        ],
        out_specs=pl.BlockSpec((1, 128), lambda i: (0, 0)),
        out_shape=jax.ShapeDtypeStruct((1, 128), jnp.float32),
        scratch_shapes=[pltpu.VMEM((1, H), jnp.float32)],
    )(*hq, W1, b1, W2p, b2p)


def kernel(x, edge_index, edge_attr, lin1_W, lin1_b, edgelin_W, edgelin_b,
           eps0, conv0_W, conv0_b, eps1, conv1_W, conv1_b, eps2, conv2_W,
           conv2_b, pred_W1, pred_b1, pred_W2, pred_b2):
    f32 = jnp.float32

    pad = E_PAD - E
    src = jnp.concatenate([edge_index[0], jnp.zeros((pad,), jnp.int32)])
    dst = jnp.concatenate([edge_index[1], jnp.full((pad,), N, jnp.int32)])
    src3 = src.reshape(NTEC, NB, EDGE_BATCH)
    dst3 = dst.reshape(NTEC, NB, EDGE_BATCH)
    zeros = jnp.zeros((N_OUT, QW), f32)

    lin1_b2 = lin1_b.reshape(1, H)
    edgelin_b2 = edgelin_b.reshape(1, H)
    W2p = jnp.zeros((H // 2, 128), f32).at[:, :O].set(pred_W2)
    b2p = jnp.zeros((1, 128), f32).at[0, :O].set(pred_b2)

    hq = _lin1(x, lin1_W, lin1_b2)
    ea_q = _edge_lin(edge_attr, edgelin_W, edgelin_b2)

    eps_row = jnp.full((1, QW), 1.0, f32) * (1.0 + eps0)
    aggq = _make_sc_round(True)(*hq, src3, dst3, zeros, *ea_q)
    hq = _conv(hq, aggq, eps_row, conv0_W, conv0_b.reshape(1, H))

    for epsk, Wk, bk in ((eps1, conv1_W, conv1_b), (eps2, conv2_W, conv2_b)):
        aggq = _make_sc_round(False)(*hq, src3, dst3, zeros, *ea_q)
        eps_row = jnp.full((1, QW), 1.0, f32) * (1.0 + epsk)
        hq = _conv(hq, aggq, eps_row, Wk, bk.reshape(1, H))

    outp = _pool_mlp(hq, pred_W1, pred_b1.reshape(1, H // 2), W2p, b2p)
    return outp[:, :O]


# speedup vs baseline: 1.1841x; 1.1841x over previous
import functools

import jax
import jax.numpy as jnp
from jax import lax
from jax.experimental import pallas as pl
from jax.experimental.pallas import tpu as pltpu
from jax.experimental.pallas import tpu_sc as plsc

N, E, F, H, ED, O = 10000, 320000, 128, 256, 16, 3
NQ = 4
QW = H // NQ
NTEC = 16
EDGE_BATCH = 128
NB = 160
E_CHUNK = NB * EDGE_BATCH
E_PAD = E_CHUNK * NTEC
EA_REAL = E // 2
ACC_SLICE = 640
N_OUT = ACC_SLICE * NTEC
ROW_BLK = 400
N_BLKS = N // ROW_BLK


def _silu(v):
    return v * jax.nn.sigmoid(v)


@functools.cache
def _make_sc_round(with_edge: bool):
    mesh = plsc.VectorSubcoreMesh(core_axis_name="c", subcore_axis_name="s",
                                  num_cores=2, num_subcores=NTEC)

    GB = 2 if with_edge else 4
    NGROUP = NB // GB
    GBE = GB * EDGE_BATCH
    out_type = [jax.ShapeDtypeStruct((N_OUT, QW), jnp.float32)
                for _ in range(NQ)]
    scratch = [
        pltpu.VMEM((GBE,), jnp.int32),
        pltpu.VMEM((GBE,), jnp.int32),
        pltpu.VMEM((GB, EDGE_BATCH), jnp.int32),
        pltpu.VMEM((GB, EDGE_BATCH), jnp.int32),
        pltpu.VMEM((GBE, QW), jnp.float32),
        pltpu.VMEM((GBE, QW), jnp.float32),
        pltpu.VMEM_SHARED((N_OUT, QW), jnp.float32),
        pltpu.SemaphoreType.DMA,
        pltpu.SemaphoreType.DMA,
        pltpu.SemaphoreType.DMA,
        pltpu.SemaphoreType.DMA,
    ]
    if with_edge:
        scratch += [
            pltpu.VMEM((EDGE_BATCH, 2 * QW), jnp.float32),
            pltpu.VMEM((EDGE_BATCH, 2 * QW), jnp.float32),
            pltpu.SemaphoreType.DMA,
            pltpu.SemaphoreType.DMA,
        ]

    def body(h0, h1, h2, h3, src2, dst3, zeros, e0, e1, e2, e3,
             a0, a1, a2, a3,
             si0, si1, di0, di1, row0, row1, acc, gs0, gs1, is0, is1,
             eav0=None, eav1=None, es0=None, es1=None):
        cid = lax.axis_index("c")
        sid = lax.axis_index("s")
        pbase = sid * (E_CHUNK // 2)
        rbase = sid * ACC_SLICE

        sidx = (si0, si1)
        didx = (di0, di1)
        rows = (row0, row1)
        eavs = (eav0, eav1)
        gsem = (gs0, gs1)
        isem = (is0, is1)
        esem = (es0, es1)

        def idx_start(g, b):
            pltpu.async_copy(src2.at[sid, pl.ds(g * GBE, GBE)],
                             sidx[b], isem[b])
            pltpu.async_copy(dst3.at[sid, pl.ds(g * GB, GB)],
                             didx[b], isem[b])

        def idx_wait(g, b):
            pltpu.make_async_copy(src2.at[sid, pl.ds(g * GBE, GBE)],
                                  sidx[b], isem[b]).wait()
            pltpu.make_async_copy(dst3.at[sid, pl.ds(g * GB, GB)],
                                  didx[b], isem[b]).wait()

        def gather_start(h_ref, b):
            pltpu.async_copy(h_ref.at[sidx[b]], rows[b], gsem[b])

        def gather_wait(h_ref, b):
            pltpu.make_async_copy(h_ref.at[sidx[b]], rows[b],
                                  gsem[b]).wait()

        def ea_slice(ea_ref, g):
            off = jnp.minimum(pbase + g * EDGE_BATCH, EA_REAL - EDGE_BATCH)
            return ea_ref.at[pl.ds(off, EDGE_BATCH)]

        def ea_start(ea_ref, g, b):
            pltpu.async_copy(ea_slice(ea_ref, g), eavs[b], esem[b])

        def ea_wait(ea_ref, g, b):
            pltpu.make_async_copy(ea_slice(ea_ref, g), eavs[b],
                                  esem[b]).wait()

        def process_group(b):
            for sub in range(GB):
                if with_edge:
                    rb, ebuf = rows[b], eavs[b]

                    def relu_row(r, carry):
                        for c in range(QW // 16):
                            sl = pl.ds(c * 16, 16)
                            el = pl.ds(sub * QW + c * 16, 16)
                            rb[sub * EDGE_BATCH + r, sl] = jnp.maximum(
                                rb[sub * EDGE_BATCH + r, sl] + ebuf[r, el],
                                0.0)
                        return carry
                    lax.fori_loop(0, EDGE_BATCH, relu_row, 0)
                pltpu.sync_copy(
                    rows[b].at[pl.ds(sub * EDGE_BATCH, EDGE_BATCH)],
                    acc.at[didx[b].at[sub]], add=True)

        def run_quarter(h_ref, ea_ref, agg_ref):
            pltpu.sync_copy(zeros.at[pl.ds(rbase, ACC_SLICE)],
                            acc.at[pl.ds(rbase, ACC_SLICE)])
            plsc.subcore_barrier()

            idx_start(0, 0)
            idx_wait(0, 0)
            gather_start(h_ref, 0)
            if with_edge:
                ea_start(ea_ref, 0, 0)
            idx_start(1, 1)

            def slot(g, b):
                @pl.when(g + 1 < NGROUP)
                def _():
                    idx_wait(g + 1, 1 - b)
                    gather_start(h_ref, 1 - b)
                    if with_edge:
                        ea_start(ea_ref, g + 1, 1 - b)
                gather_wait(h_ref, b)
                if with_edge:
                    ea_wait(ea_ref, g, b)
                process_group(b)

                @pl.when(g + 2 < NGROUP)
                def _():
                    idx_start(g + 2, b)

            def step(i, carry):
                slot(2 * i, 0)
                slot(2 * i + 1, 1)
                return carry
            lax.fori_loop(0, NGROUP // 2, step, 0)
            plsc.subcore_barrier()
            pltpu.sync_copy(acc.at[pl.ds(rbase, ACC_SLICE)],
                            agg_ref.at[pl.ds(rbase, ACC_SLICE)])
            plsc.subcore_barrier()

        @pl.when(cid == 0)
        def _():
            run_quarter(h0, e0, a0)
            run_quarter(h1, e1, a1)

        @pl.when(cid == 1)
        def _():
            run_quarter(h2, e2, a2)
            run_quarter(h3, e3, a3)

    return pl.kernel(
        body, out_type=out_type, mesh=mesh, scratch_types=scratch,
        compiler_params=pltpu.CompilerParams(use_tc_tiling_on_sc=False))


def _lin1_body(x, W, b, *hq):
    h = _silu(_silu(jnp.dot(x[...], W[...],
                            preferred_element_type=jnp.float32) + b[...]))
    for q in range(NQ):
        hq[q][...] = h[:, q * QW:(q + 1) * QW]


def _lin1(x, W, b):
    return pl.pallas_call(
        _lin1_body,
        grid=(N_BLKS,),
        in_specs=[
            pl.BlockSpec((ROW_BLK, F), lambda i: (i, 0)),
            pl.BlockSpec((F, H), lambda i: (0, 0)),
            pl.BlockSpec((1, H), lambda i: (0, 0)),
        ],
        out_specs=[pl.BlockSpec((ROW_BLK, QW), lambda i: (i, 0))] * NQ,
        out_shape=[jax.ShapeDtypeStruct((N, QW), jnp.float32)] * NQ,
    )(x, W, b)


_EA_BLK = 1280
_EA_BLKS = E // _EA_BLK


def _ea_body(xe, W, b, *eo):
    v = jnp.dot(xe[...], W[...], preferred_element_type=jnp.float32) + b[...]
    for q in range(NQ):
        qc = v[:, q * QW:(q + 1) * QW]
        strips = [jnp.concatenate(
            [qc[256 * s:256 * s + 128, :], qc[256 * s + 128:256 * (s + 1), :]],
            axis=1) for s in range(_EA_BLK // 256)]
        eo[q][...] = jnp.concatenate(strips, axis=0)


def _edge_lin(edge_attr, W, b):
    return pl.pallas_call(
        _ea_body,
        grid=(_EA_BLKS,),
        in_specs=[
            pl.BlockSpec((_EA_BLK, ED), lambda i: (i, 0)),
            pl.BlockSpec((ED, H), lambda i: (0, 0)),
            pl.BlockSpec((1, H), lambda i: (0, 0)),
        ],
        out_specs=[pl.BlockSpec((_EA_BLK // 2, 2 * QW), lambda i: (i, 0))] * NQ,
        out_shape=[jax.ShapeDtypeStruct((EA_REAL, 2 * QW), jnp.float32)] * NQ,
    )(edge_attr, W, b)


def _conv_body(*refs):
    hq = refs[0:NQ]
    aq = refs[NQ:2 * NQ]
    epsr, W, b = refs[2 * NQ:2 * NQ + 3]
    oq = refs[2 * NQ + 3:]
    acc = None
    for q in range(NQ):
        z = epsr[...] * hq[q][...] + aq[q][...]
        p = jnp.dot(z, W[q * QW:(q + 1) * QW, :],
                    preferred_element_type=jnp.float32)
        acc = p if acc is None else acc + p
    h = _silu(acc + b[...])
    for q in range(NQ):
        oq[q][...] = h[:, q * QW:(q + 1) * QW]


def _conv(hq, aggq, eps_row, W, b):
    return pl.pallas_call(
        _conv_body,
        grid=(N_BLKS,),
        in_specs=(
            [pl.BlockSpec((ROW_BLK, QW), lambda i: (i, 0))] * NQ
            + [pl.BlockSpec((ROW_BLK, QW), lambda i: (i, 0))] * NQ
            + [
                pl.BlockSpec((1, QW), lambda i: (0, 0)),
                pl.BlockSpec((H, H), lambda i: (0, 0)),
                pl.BlockSpec((1, H), lambda i: (0, 0)),
            ]
        ),
        out_specs=[pl.BlockSpec((ROW_BLK, QW), lambda i: (i, 0))] * NQ,
        out_shape=[jax.ShapeDtypeStruct((N, QW), jnp.float32)] * NQ,
    )(*hq, *aggq, eps_row, W, b)


def _pool_body(h0, h1, h2, h3, W1, b1, W2, b2, out, acc):
    i = pl.program_id(0)

    @pl.when(i == 0)
    def _():
        acc[...] = jnp.zeros_like(acc)

    blk = jnp.concatenate([h0[...], h1[...], h2[...], h3[...]], axis=1)
    acc[...] += jnp.sum(blk, axis=0, keepdims=True)

    @pl.when(i == N_BLKS - 1)
    def _():
        g = _silu(acc[...])
        p = _silu(jnp.dot(g, W1[...], preferred_element_type=jnp.float32)
                  + b1[...])
        out[...] = jnp.dot(p, W2[...], preferred_element_type=jnp.float32) \
            + b2[...]


def _pool_mlp(hq, W1, b1, W2p, b2p):
    return pl.pallas_call(
        _pool_body,
        grid=(N_BLKS,),
        in_specs=[pl.BlockSpec((ROW_BLK, QW), lambda i: (i, 0))] * NQ + [
            pl.BlockSpec((H, H // 2), lambda i: (0, 0)),
            pl.BlockSpec((1, H // 2), lambda i: (0, 0)),
            pl.BlockSpec((H // 2, 128), lambda i: (0, 0)),
            pl.BlockSpec((1, 128), lambda i: (0, 0)),
        ],
        out_specs=pl.BlockSpec((1, 128), lambda i: (0, 0)),
        out_shape=jax.ShapeDtypeStruct((1, 128), jnp.float32),
        scratch_shapes=[pltpu.VMEM((1, H), jnp.float32)],
    )(*hq, W1, b1, W2p, b2p)


def kernel(x, edge_index, edge_attr, lin1_W, lin1_b, edgelin_W, edgelin_b,
           eps0, conv0_W, conv0_b, eps1, conv1_W, conv1_b, eps2, conv2_W,
           conv2_b, pred_W1, pred_b1, pred_W2, pred_b2):
    f32 = jnp.float32

    pad = E_PAD - E
    src = jnp.concatenate([edge_index[0], jnp.zeros((pad,), jnp.int32)])
    dst = jnp.concatenate([edge_index[1], jnp.full((pad,), N, jnp.int32)])
    src2 = src.reshape(NTEC, E_CHUNK)
    dst3 = dst.reshape(NTEC, NB, EDGE_BATCH)
    zeros = jnp.zeros((N_OUT, QW), f32)

    lin1_b2 = lin1_b.reshape(1, H)
    edgelin_b2 = edgelin_b.reshape(1, H)
    W2p = jnp.zeros((H // 2, 128), f32).at[:, :O].set(pred_W2)
    b2p = jnp.zeros((1, 128), f32).at[0, :O].set(pred_b2)

    hq = _lin1(x, lin1_W, lin1_b2)
    ea_q = _edge_lin(edge_attr, edgelin_W, edgelin_b2)

    eps_row = jnp.full((1, QW), 1.0, f32) * (1.0 + eps0)
    aggq = _make_sc_round(True)(*hq, src2, dst3, zeros, *ea_q)
    hq = _conv(hq, aggq, eps_row, conv0_W, conv0_b.reshape(1, H))

    for epsk, Wk, bk in ((eps1, conv1_W, conv1_b), (eps2, conv2_W, conv2_b)):
        aggq = _make_sc_round(False)(*hq, src2, dst3, zeros, *ea_q)
        eps_row = jnp.full((1, QW), 1.0, f32) * (1.0 + epsk)
        hq = _conv(hq, aggq, eps_row, Wk, bk.reshape(1, H))

    outp = _pool_mlp(hq, pred_W1, pred_b1.reshape(1, H // 2), W2p, b2p)
    return outp[:, :O]


# speedup vs baseline: 1.7115x; 1.4454x over previous
import functools

import jax
import jax.numpy as jnp
from jax import lax
from jax.experimental import pallas as pl
from jax.experimental.pallas import tpu as pltpu
from jax.experimental.pallas import tpu_sc as plsc

N, E, F, H, ED, O = 10000, 320000, 128, 256, 16, 3
NQ = 4
QW = H // NQ
NTEC = 16
EDGE_BATCH = 128
NB = 160
E_CHUNK = NB * EDGE_BATCH
E_PAD = E_CHUNK * NTEC
EA_REAL = E // 2
ACC_SLICE = 640
N_OUT = ACC_SLICE * NTEC
ROW_BLK = 400
N_BLKS = N // ROW_BLK


def _silu(v):
    return v * jax.nn.sigmoid(v)


@functools.cache
def _make_sc_round(with_edge: bool):
    mesh = plsc.VectorSubcoreMesh(core_axis_name="c", subcore_axis_name="s",
                                  num_cores=2, num_subcores=NTEC)

    GB = 2
    NGROUP = NB // GB
    GBE = GB * EDGE_BATCH
    UNROLL = 12
    SLOTS = 84
    out_type = [jax.ShapeDtypeStruct((N_OUT, QW), jnp.float32)
                for _ in range(NQ)]
    scratch = [
        [pltpu.VMEM((GBE,), jnp.int32)] * 4,
        [pltpu.VMEM((GB, EDGE_BATCH), jnp.int32)] * 4,
        [pltpu.VMEM((GBE, QW), jnp.float32)] * 3,
        pltpu.VMEM_SHARED((N_OUT, QW), jnp.float32),
        [pltpu.SemaphoreType.DMA] * 3,
        [pltpu.SemaphoreType.DMA] * 3,
        [pltpu.SemaphoreType.DMA] * 4,
    ]
    if with_edge:
        scratch += [
            [pltpu.VMEM((EDGE_BATCH, 2 * QW), jnp.float32)] * 2,
            [pltpu.SemaphoreType.DMA] * 2,
        ]

    def body(h0, h1, h2, h3, srcf, dst2, zeros, e0, e1, e2, e3,
             a0, a1, a2, a3,
             sidx, didx, rows, acc, gsem, ssem, isem,
             eavs=None, esem=None):
        cid = lax.axis_index("c")
        sid = lax.axis_index("s")
        ebase = sid * E_CHUNK
        pbase = sid * (E_CHUNK // 2)
        rbase = sid * ACC_SLICE
        ng = jnp.minimum(NGROUP, (E - ebase) // GBE)

        def idx_start(g, x):
            pltpu.async_copy(srcf.at[pl.ds(ebase + g * GBE, GBE)],
                             sidx[x], isem[x])
            pltpu.async_copy(
                dst2.at[pl.ds((ebase + g * GBE) // EDGE_BATCH, GB)],
                didx[x], isem[x])

        def idx_wait(g, x):
            pltpu.make_async_copy(srcf.at[pl.ds(ebase + g * GBE, GBE)],
                                  sidx[x], isem[x]).wait()
            pltpu.make_async_copy(
                dst2.at[pl.ds((ebase + g * GBE) // EDGE_BATCH, GB)],
                didx[x], isem[x]).wait()

        def gather_start(h_ref, r, x):
            pltpu.async_copy(h_ref.at[sidx[x]], rows[r], gsem[r])

        def gather_wait(h_ref, r, x):
            pltpu.make_async_copy(h_ref.at[sidx[x]], rows[r],
                                  gsem[r]).wait()

        def ea_slice(ea_ref, g):
            return ea_ref.at[pl.ds(pbase + g * EDGE_BATCH, EDGE_BATCH)]

        def ea_start(ea_ref, g, e):
            pltpu.async_copy(ea_slice(ea_ref, g), eavs[e], esem[e])

        def ea_wait(ea_ref, g, e):
            pltpu.make_async_copy(ea_slice(ea_ref, g), eavs[e],
                                  esem[e]).wait()

        def relu_group(r, e):
            rb, ebuf = rows[r], eavs[e]

            def relu_row(i, carry):
                for sub in range(GB):
                    for c in range(QW // 16):
                        sl = pl.ds(c * 16, 16)
                        el = pl.ds(sub * QW + c * 16, 16)
                        rb[sub * EDGE_BATCH + i, sl] = jnp.maximum(
                            rb[sub * EDGE_BATCH + i, sl] + ebuf[i, el], 0.0)
                return carry
            lax.fori_loop(0, EDGE_BATCH, relu_row, 0)

        def scatter_group(r, x):
            for sub in range(GB):
                pltpu.async_copy(
                    rows[r].at[pl.ds(sub * EDGE_BATCH, EDGE_BATCH)],
                    acc.at[didx[x].at[sub]], ssem[r], add=True)

        def scatter_drain(r, x):
            for sub in range(GB):
                pltpu.make_async_copy(
                    rows[r].at[pl.ds(sub * EDGE_BATCH, EDGE_BATCH)],
                    acc.at[didx[x].at[sub]], ssem[r]).wait()

        def run_quarter(h_ref, ea_ref, agg_ref):
            pltpu.sync_copy(zeros.at[pl.ds(rbase, ACC_SLICE)],
                            acc.at[pl.ds(rbase, ACC_SLICE)])
            plsc.subcore_barrier()

            idx_start(0, 0)
            idx_wait(0, 0)
            gather_start(h_ref, 0, 0)
            if with_edge:
                ea_start(ea_ref, 0, 0)
            idx_start(1, 1)

            def slot(g, s):
                @pl.when(jnp.logical_and(g >= 2, g - 2 < ng))
                def _():
                    scatter_drain((s - 2) % 3, (s - 2) % 4)

                @pl.when(g + 1 < ng)
                def _():
                    idx_wait(g + 1, (s + 1) % 4)
                    gather_start(h_ref, (s + 1) % 3, (s + 1) % 4)
                    if with_edge:
                        ea_start(ea_ref, g + 1, (s + 1) % 2)

                @pl.when(g < ng)
                def _():
                    gather_wait(h_ref, s % 3, s % 4)
                    if with_edge:
                        ea_wait(ea_ref, g, s % 2)
                        relu_group(s % 3, s % 2)
                    scatter_group(s % 3, s % 4)

                @pl.when(g + 2 < ng)
                def _():
                    idx_start(g + 2, (s + 2) % 4)

            def step(i, carry):
                for s in range(UNROLL):
                    slot(UNROLL * i + s, s)
                return carry
            lax.fori_loop(0, SLOTS // UNROLL, step, 0)
            plsc.subcore_barrier()
            pltpu.sync_copy(acc.at[pl.ds(rbase, ACC_SLICE)],
                            agg_ref.at[pl.ds(rbase, ACC_SLICE)])
            plsc.subcore_barrier()

        @pl.when(cid == 0)
        def _():
            run_quarter(h0, e0, a0)
            run_quarter(h1, e1, a1)

        @pl.when(cid == 1)
        def _():
            run_quarter(h2, e2, a2)
            run_quarter(h3, e3, a3)

    return pl.kernel(
        body, out_type=out_type, mesh=mesh, scratch_types=scratch,
        compiler_params=pltpu.CompilerParams(use_tc_tiling_on_sc=False))


def _lin1_body(x, W, b, *hq):
    h = _silu(_silu(jnp.dot(x[...], W[...],
                            preferred_element_type=jnp.float32) + b[...]))
    for q in range(NQ):
        hq[q][...] = h[:, q * QW:(q + 1) * QW]


def _lin1(x, W, b):
    return pl.pallas_call(
        _lin1_body,
        grid=(N_BLKS,),
        in_specs=[
            pl.BlockSpec((ROW_BLK, F), lambda i: (i, 0)),
            pl.BlockSpec((F, H), lambda i: (0, 0)),
            pl.BlockSpec((1, H), lambda i: (0, 0)),
        ],
        out_specs=[pl.BlockSpec((ROW_BLK, QW), lambda i: (i, 0))] * NQ,
        out_shape=[jax.ShapeDtypeStruct((N, QW), jnp.float32)] * NQ,
    )(x, W, b)


_EA_BLK = 1280
_EA_BLKS = E // _EA_BLK


def _ea_body(xe, W, b, *eo):
    v = jnp.dot(xe[...], W[...], preferred_element_type=jnp.float32) + b[...]
    for q in range(NQ):
        qc = v[:, q * QW:(q + 1) * QW]
        strips = [jnp.concatenate(
            [qc[256 * s:256 * s + 128, :], qc[256 * s + 128:256 * (s + 1), :]],
            axis=1) for s in range(_EA_BLK // 256)]
        eo[q][...] = jnp.concatenate(strips, axis=0)


def _edge_lin(edge_attr, W, b):
    return pl.pallas_call(
        _ea_body,
        grid=(_EA_BLKS,),
        in_specs=[
            pl.BlockSpec((_EA_BLK, ED), lambda i: (i, 0)),
            pl.BlockSpec((ED, H), lambda i: (0, 0)),
            pl.BlockSpec((1, H), lambda i: (0, 0)),
        ],
        out_specs=[pl.BlockSpec((_EA_BLK // 2, 2 * QW), lambda i: (i, 0))] * NQ,
        out_shape=[jax.ShapeDtypeStruct((EA_REAL, 2 * QW), jnp.float32)] * NQ,
    )(edge_attr, W, b)


def _conv_body(*refs):
    hq = refs[0:NQ]
    aq = refs[NQ:2 * NQ]
    epsr, W, b = refs[2 * NQ:2 * NQ + 3]
    oq = refs[2 * NQ + 3:]
    acc = None
    for q in range(NQ):
        z = epsr[...] * hq[q][...] + aq[q][...]
        p = jnp.dot(z, W[q * QW:(q + 1) * QW, :],
                    preferred_element_type=jnp.float32)
        acc = p if acc is None else acc + p
    h = _silu(acc + b[...])
    for q in range(NQ):
        oq[q][...] = h[:, q * QW:(q + 1) * QW]


def _conv(hq, aggq, eps_row, W, b):
    return pl.pallas_call(
        _conv_body,
        grid=(N_BLKS,),
        in_specs=(
            [pl.BlockSpec((ROW_BLK, QW), lambda i: (i, 0))] * NQ
            + [pl.BlockSpec((ROW_BLK, QW), lambda i: (i, 0))] * NQ
            + [
                pl.BlockSpec((1, QW), lambda i: (0, 0)),
                pl.BlockSpec((H, H), lambda i: (0, 0)),
                pl.BlockSpec((1, H), lambda i: (0, 0)),
            ]
        ),
        out_specs=[pl.BlockSpec((ROW_BLK, QW), lambda i: (i, 0))] * NQ,
        out_shape=[jax.ShapeDtypeStruct((N, QW), jnp.float32)] * NQ,
    )(*hq, *aggq, eps_row, W, b)


def _pool_body(h0, h1, h2, h3, W1, b1, W2, b2, out, acc):
    i = pl.program_id(0)

    @pl.when(i == 0)
    def _():
        acc[...] = jnp.zeros_like(acc)

    blk = jnp.concatenate([h0[...], h1[...], h2[...], h3[...]], axis=1)
    acc[...] += jnp.sum(blk, axis=0, keepdims=True)

    @pl.when(i == N_BLKS - 1)
    def _():
        g = _silu(acc[...])
        p = _silu(jnp.dot(g, W1[...], preferred_element_type=jnp.float32)
                  + b1[...])
        out[...] = jnp.dot(p, W2[...], preferred_element_type=jnp.float32) \
            + b2[...]


def _pool_mlp(hq, W1, b1, W2p, b2p):
    return pl.pallas_call(
        _pool_body,
        grid=(N_BLKS,),
        in_specs=[pl.BlockSpec((ROW_BLK, QW), lambda i: (i, 0))] * NQ + [
            pl.BlockSpec((H, H // 2), lambda i: (0, 0)),
            pl.BlockSpec((1, H // 2), lambda i: (0, 0)),
            pl.BlockSpec((H // 2, 128), lambda i: (0, 0)),
            pl.BlockSpec((1, 128), lambda i: (0, 0)),
        ],
        out_specs=pl.BlockSpec((1, 128), lambda i: (0, 0)),
        out_shape=jax.ShapeDtypeStruct((1, 128), jnp.float32),
        scratch_shapes=[pltpu.VMEM((1, H), jnp.float32)],
    )(*hq, W1, b1, W2p, b2p)


def kernel(x, edge_index, edge_attr, lin1_W, lin1_b, edgelin_W, edgelin_b,
           eps0, conv0_W, conv0_b, eps1, conv1_W, conv1_b, eps2, conv2_W,
           conv2_b, pred_W1, pred_b1, pred_W2, pred_b2):
    f32 = jnp.float32

    srcf = edge_index[0]
    dst2 = edge_index[1].reshape(E // EDGE_BATCH, EDGE_BATCH)
    zeros = jnp.zeros((N_OUT, QW), f32)

    lin1_b2 = lin1_b.reshape(1, H)
    edgelin_b2 = edgelin_b.reshape(1, H)
    W2p = jnp.zeros((H // 2, 128), f32).at[:, :O].set(pred_W2)
    b2p = jnp.zeros((1, 128), f32).at[0, :O].set(pred_b2)

    hq = _lin1(x, lin1_W, lin1_b2)
    ea_q = _edge_lin(edge_attr, edgelin_W, edgelin_b2)

    eps_row = jnp.full((1, QW), 1.0, f32) * (1.0 + eps0)
    aggq = _make_sc_round(True)(*hq, srcf, dst2, zeros, *ea_q)
    hq = _conv(hq, aggq, eps_row, conv0_W, conv0_b.reshape(1, H))

    for epsk, Wk, bk in ((eps1, conv1_W, conv1_b), (eps2, conv2_W, conv2_b)):
        aggq = _make_sc_round(False)(*hq, srcf, dst2, zeros, *ea_q)
        eps_row = jnp.full((1, QW), 1.0, f32) * (1.0 + epsk)
        hq = _conv(hq, aggq, eps_row, Wk, bk.reshape(1, H))

    outp = _pool_mlp(hq, pred_W1, pred_b1.reshape(1, H // 2), W2p, b2p)
    return outp[:, :O]


# speedup vs baseline: 1.7118x; 1.0002x over previous
import functools

import jax
import jax.numpy as jnp
from jax import lax
from jax.experimental import pallas as pl
from jax.experimental.pallas import tpu as pltpu
from jax.experimental.pallas import tpu_sc as plsc

N, E, F, H, ED, O = 10000, 320000, 128, 256, 16, 3
NQ = 4
QW = H // NQ
NTEC = 16
EDGE_BATCH = 128
NB = 160
E_CHUNK = NB * EDGE_BATCH
E_PAD = E_CHUNK * NTEC
EA_REAL = E // 2
ACC_SLICE = 640
N_OUT = ACC_SLICE * NTEC
ROW_BLK = 400
N_BLKS = N // ROW_BLK


def _silu(v):
    return v * jax.nn.sigmoid(v)


@functools.cache
def _make_sc_round(with_edge: bool):
    mesh = plsc.VectorSubcoreMesh(core_axis_name="c", subcore_axis_name="s",
                                  num_cores=2, num_subcores=NTEC)

    GB = 2
    NGROUP = NB // GB
    GBE = GB * EDGE_BATCH
    UNROLL = 12
    SLOTS = 84
    out_type = [jax.ShapeDtypeStruct((N_OUT, QW), jnp.float32)
                for _ in range(NQ)]
    scratch = [
        [pltpu.VMEM((GBE,), jnp.int32)] * 4,
        [pltpu.VMEM((GB, EDGE_BATCH), jnp.int32)] * 4,
        [pltpu.VMEM((GBE, QW), jnp.float32)] * 3,
        pltpu.VMEM_SHARED((N_OUT, QW), jnp.float32),
        [pltpu.SemaphoreType.DMA] * 3,
        [pltpu.SemaphoreType.DMA] * 3,
        [pltpu.SemaphoreType.DMA] * 4,
    ]
    if with_edge:
        scratch += [
            [pltpu.VMEM((EDGE_BATCH, 2 * QW), jnp.float32)] * 2,
            [pltpu.SemaphoreType.DMA] * 2,
        ]

    def body(h0, h1, h2, h3, srcf, dst2, zeros, e0, e1, e2, e3,
             a0, a1, a2, a3,
             sidx, didx, rows, acc, gsem, ssem, isem,
             eavs=None, esem=None):
        cid = lax.axis_index("c")
        sid = lax.axis_index("s")
        ebase = sid * E_CHUNK
        pbase = sid * (E_CHUNK // 2)
        rbase = sid * ACC_SLICE
        ng = jnp.minimum(NGROUP, (E - ebase) // GBE)

        def idx_start(g, x):
            pltpu.async_copy(srcf.at[pl.ds(ebase + g * GBE, GBE)],
                             sidx[x], isem[x])
            pltpu.async_copy(
                dst2.at[pl.ds((ebase + g * GBE) // EDGE_BATCH, GB)],
                didx[x], isem[x])

        def idx_wait(g, x):
            pltpu.make_async_copy(srcf.at[pl.ds(ebase + g * GBE, GBE)],
                                  sidx[x], isem[x]).wait()
            pltpu.make_async_copy(
                dst2.at[pl.ds((ebase + g * GBE) // EDGE_BATCH, GB)],
                didx[x], isem[x]).wait()

        def gather_start(h_ref, r, x):
            pltpu.async_copy(h_ref.at[sidx[x]], rows[r], gsem[r])

        def gather_wait(h_ref, r, x):
            pltpu.make_async_copy(h_ref.at[sidx[x]], rows[r],
                                  gsem[r]).wait()

        def ea_slice(ea_ref, g):
            return ea_ref.at[pl.ds(pbase + g * EDGE_BATCH, EDGE_BATCH)]

        def ea_start(ea_ref, g, e):
            pltpu.async_copy(ea_slice(ea_ref, g), eavs[e], esem[e])

        def ea_wait(ea_ref, g, e):
            pltpu.make_async_copy(ea_slice(ea_ref, g), eavs[e],
                                  esem[e]).wait()

        def relu_group(r, e):
            rb, ebuf = rows[r], eavs[e]
            RU = 2

            def relu_row(i, carry):
                for u in range(RU):
                    row = RU * i + u
                    for sub in range(GB):
                        for c in range(QW // 16):
                            sl = pl.ds(c * 16, 16)
                            el = pl.ds(sub * QW + c * 16, 16)
                            rb[sub * EDGE_BATCH + row, sl] = jnp.maximum(
                                rb[sub * EDGE_BATCH + row, sl]
                                + ebuf[row, el], 0.0)
                return carry
            lax.fori_loop(0, EDGE_BATCH // RU, relu_row, 0)

        def scatter_group(r, x):
            for sub in range(GB):
                pltpu.async_copy(
                    rows[r].at[pl.ds(sub * EDGE_BATCH, EDGE_BATCH)],
                    acc.at[didx[x].at[sub]], ssem[r], add=True)

        def scatter_drain(r, x):
            for sub in range(GB):
                pltpu.make_async_copy(
                    rows[r].at[pl.ds(sub * EDGE_BATCH, EDGE_BATCH)],
                    acc.at[didx[x].at[sub]], ssem[r]).wait()

        def run_quarter(h_ref, ea_ref, agg_ref):
            pltpu.sync_copy(zeros.at[pl.ds(rbase, ACC_SLICE)],
                            acc.at[pl.ds(rbase, ACC_SLICE)])
            plsc.subcore_barrier()

            idx_start(0, 0)
            idx_wait(0, 0)
            gather_start(h_ref, 0, 0)
            if with_edge:
                ea_start(ea_ref, 0, 0)
            idx_start(1, 1)

            def slot(g, s):
                @pl.when(jnp.logical_and(g >= 2, g - 2 < ng))
                def _():
                    scatter_drain((s - 2) % 3, (s - 2) % 4)

                @pl.when(g + 1 < ng)
                def _():
                    idx_wait(g + 1, (s + 1) % 4)
                    gather_start(h_ref, (s + 1) % 3, (s + 1) % 4)
                    if with_edge:
                        ea_start(ea_ref, g + 1, (s + 1) % 2)

                @pl.when(g < ng)
                def _():
                    gather_wait(h_ref, s % 3, s % 4)
                    if with_edge:
                        ea_wait(ea_ref, g, s % 2)
                        relu_group(s % 3, s % 2)
                    scatter_group(s % 3, s % 4)

                @pl.when(g + 2 < ng)
                def _():
                    idx_start(g + 2, (s + 2) % 4)

            def step(i, carry):
                for s in range(UNROLL):
                    slot(UNROLL * i + s, s)
                return carry
            lax.fori_loop(0, SLOTS // UNROLL, step, 0)
            plsc.subcore_barrier()
            pltpu.sync_copy(acc.at[pl.ds(rbase, ACC_SLICE)],
                            agg_ref.at[pl.ds(rbase, ACC_SLICE)])
            plsc.subcore_barrier()

        @pl.when(cid == 0)
        def _():
            run_quarter(h0, e0, a0)
            run_quarter(h1, e1, a1)

        @pl.when(cid == 1)
        def _():
            run_quarter(h2, e2, a2)
            run_quarter(h3, e3, a3)

    return pl.kernel(
        body, out_type=out_type, mesh=mesh, scratch_types=scratch,
        compiler_params=pltpu.CompilerParams(use_tc_tiling_on_sc=False))


def _lin1_body(x, W, b, *hq):
    h = _silu(_silu(jnp.dot(x[...], W[...],
                            preferred_element_type=jnp.float32) + b[...]))
    for q in range(NQ):
        hq[q][...] = h[:, q * QW:(q + 1) * QW]


def _lin1(x, W, b):
    return pl.pallas_call(
        _lin1_body,
        grid=(N_BLKS,),
        in_specs=[
            pl.BlockSpec((ROW_BLK, F), lambda i: (i, 0)),
            pl.BlockSpec((F, H), lambda i: (0, 0)),
            pl.BlockSpec((1, H), lambda i: (0, 0)),
        ],
        out_specs=[pl.BlockSpec((ROW_BLK, QW), lambda i: (i, 0))] * NQ,
        out_shape=[jax.ShapeDtypeStruct((N, QW), jnp.float32)] * NQ,
    )(x, W, b)


_EA_BLK = 1280
_EA_BLKS = E // _EA_BLK


def _ea_body(xe, W, b, *eo):
    v = jnp.dot(xe[...], W[...], preferred_element_type=jnp.float32) + b[...]
    for q in range(NQ):
        qc = v[:, q * QW:(q + 1) * QW]
        strips = [jnp.concatenate(
            [qc[256 * s:256 * s + 128, :], qc[256 * s + 128:256 * (s + 1), :]],
            axis=1) for s in range(_EA_BLK // 256)]
        eo[q][...] = jnp.concatenate(strips, axis=0)


def _edge_lin(edge_attr, W, b):
    return pl.pallas_call(
        _ea_body,
        grid=(_EA_BLKS,),
        in_specs=[
            pl.BlockSpec((_EA_BLK, ED), lambda i: (i, 0)),
            pl.BlockSpec((ED, H), lambda i: (0, 0)),
            pl.BlockSpec((1, H), lambda i: (0, 0)),
        ],
        out_specs=[pl.BlockSpec((_EA_BLK // 2, 2 * QW), lambda i: (i, 0))] * NQ,
        out_shape=[jax.ShapeDtypeStruct((EA_REAL, 2 * QW), jnp.float32)] * NQ,
    )(edge_attr, W, b)


def _conv_body(*refs):
    hq = refs[0:NQ]
    aq = refs[NQ:2 * NQ]
    epsr, W, b = refs[2 * NQ:2 * NQ + 3]
    oq = refs[2 * NQ + 3:]
    acc = None
    for q in range(NQ):
        z = epsr[...] * hq[q][...] + aq[q][...]
        p = jnp.dot(z, W[q * QW:(q + 1) * QW, :],
                    preferred_element_type=jnp.float32)
        acc = p if acc is None else acc + p
    h = _silu(acc + b[...])
    for q in range(NQ):
        oq[q][...] = h[:, q * QW:(q + 1) * QW]


def _conv(hq, aggq, eps_row, W, b):
    return pl.pallas_call(
        _conv_body,
        grid=(N_BLKS,),
        in_specs=(
            [pl.BlockSpec((ROW_BLK, QW), lambda i: (i, 0))] * NQ
            + [pl.BlockSpec((ROW_BLK, QW), lambda i: (i, 0))] * NQ
            + [
                pl.BlockSpec((1, QW), lambda i: (0, 0)),
                pl.BlockSpec((H, H), lambda i: (0, 0)),
                pl.BlockSpec((1, H), lambda i: (0, 0)),
            ]
        ),
        out_specs=[pl.BlockSpec((ROW_BLK, QW), lambda i: (i, 0))] * NQ,
        out_shape=[jax.ShapeDtypeStruct((N, QW), jnp.float32)] * NQ,
    )(*hq, *aggq, eps_row, W, b)


def _pool_body(h0, h1, h2, h3, W1, b1, W2, b2, out, acc):
    i = pl.program_id(0)

    @pl.when(i == 0)
    def _():
        acc[...] = jnp.zeros_like(acc)

    blk = jnp.concatenate([h0[...], h1[...], h2[...], h3[...]], axis=1)
    acc[...] += jnp.sum(blk, axis=0, keepdims=True)

    @pl.when(i == N_BLKS - 1)
    def _():
        g = _silu(acc[...])
        p = _silu(jnp.dot(g, W1[...], preferred_element_type=jnp.float32)
                  + b1[...])
        out[...] = jnp.dot(p, W2[...], preferred_element_type=jnp.float32) \
            + b2[...]


def _pool_mlp(hq, W1, b1, W2p, b2p):
    return pl.pallas_call(
        _pool_body,
        grid=(N_BLKS,),
        in_specs=[pl.BlockSpec((ROW_BLK, QW), lambda i: (i, 0))] * NQ + [
            pl.BlockSpec((H, H // 2), lambda i: (0, 0)),
            pl.BlockSpec((1, H // 2), lambda i: (0, 0)),
            pl.BlockSpec((H // 2, 128), lambda i: (0, 0)),
            pl.BlockSpec((1, 128), lambda i: (0, 0)),
        ],
        out_specs=pl.BlockSpec((1, 128), lambda i: (0, 0)),
        out_shape=jax.ShapeDtypeStruct((1, 128), jnp.float32),
        scratch_shapes=[pltpu.VMEM((1, H), jnp.float32)],
    )(*hq, W1, b1, W2p, b2p)


def kernel(x, edge_index, edge_attr, lin1_W, lin1_b, edgelin_W, edgelin_b,
           eps0, conv0_W, conv0_b, eps1, conv1_W, conv1_b, eps2, conv2_W,
           conv2_b, pred_W1, pred_b1, pred_W2, pred_b2):
    f32 = jnp.float32

    srcf = edge_index[0]
    dst2 = edge_index[1].reshape(E // EDGE_BATCH, EDGE_BATCH)
    zeros = jnp.zeros((N_OUT, QW), f32)

    lin1_b2 = lin1_b.reshape(1, H)
    edgelin_b2 = edgelin_b.reshape(1, H)
    W2p = jnp.zeros((H // 2, 128), f32).at[:, :O].set(pred_W2)
    b2p = jnp.zeros((1, 128), f32).at[0, :O].set(pred_b2)

    hq = _lin1(x, lin1_W, lin1_b2)
    ea_q = _edge_lin(edge_attr, edgelin_W, edgelin_b2)

    eps_row = jnp.full((1, QW), 1.0, f32) * (1.0 + eps0)
    aggq = _make_sc_round(True)(*hq, srcf, dst2, zeros, *ea_q)
    hq = _conv(hq, aggq, eps_row, conv0_W, conv0_b.reshape(1, H))

    for epsk, Wk, bk in ((eps1, conv1_W, conv1_b), (eps2, conv2_W, conv2_b)):
        aggq = _make_sc_round(False)(*hq, srcf, dst2, zeros, *ea_q)
        eps_row = jnp.full((1, QW), 1.0, f32) * (1.0 + epsk)
        hq = _conv(hq, aggq, eps_row, Wk, bk.reshape(1, H))

    outp = _pool_mlp(hq, pred_W1, pred_b1.reshape(1, H // 2), W2p, b2p)
    return outp[:, :O]


# speedup vs baseline: 1.7525x; 1.0238x over previous
import functools

import jax
import jax.numpy as jnp
from jax import lax
from jax.experimental import pallas as pl
from jax.experimental.pallas import tpu as pltpu
from jax.experimental.pallas import tpu_sc as plsc

N, E, F, H, ED, O = 10000, 320000, 128, 256, 16, 3
NQ = 4
QW = H // NQ
NTEC = 16
EDGE_BATCH = 128
NB = 160
E_CHUNK = NB * EDGE_BATCH
E_PAD = E_CHUNK * NTEC
EA_REAL = E // 2
ACC_SLICE = 640
N_OUT = ACC_SLICE * NTEC
ROW_BLK = 400
N_BLKS = N // ROW_BLK


def _silu(v):
    return v * jax.nn.sigmoid(v)


@functools.cache
def _make_sc_round(with_edge: bool, nq: int = NQ):
    mesh = plsc.VectorSubcoreMesh(core_axis_name="c", subcore_axis_name="s",
                                  num_cores=2, num_subcores=NTEC)

    GB = 2
    NGROUP = NB // GB
    GBE = GB * EDGE_BATCH
    UNROLL = 12
    SLOTS = 84
    out_type = [jax.ShapeDtypeStruct((N_OUT, QW), jnp.float32)
                for _ in range(nq)]
    scratch = [
        [pltpu.VMEM((GBE,), jnp.int32)] * 4,
        [pltpu.VMEM((GB, EDGE_BATCH), jnp.int32)] * 4,
        [pltpu.VMEM((GBE, QW), jnp.float32)] * 3,
        pltpu.VMEM_SHARED((N_OUT, QW), jnp.float32),
        [pltpu.SemaphoreType.DMA] * 3,
        [pltpu.SemaphoreType.DMA] * 3,
        [pltpu.SemaphoreType.DMA] * 4,
    ]
    if with_edge:
        scratch += [
            [pltpu.VMEM((EDGE_BATCH, 2 * QW), jnp.float32)] * 2,
            [pltpu.SemaphoreType.DMA] * 2,
        ]

    def body(*args):
        hs = args[0:nq]
        srcf, dst2, zeros = args[nq:nq + 3]
        es = args[nq + 3:2 * nq + 3]
        aggs = args[2 * nq + 3:3 * nq + 3]
        scr = args[3 * nq + 3:]
        sidx, didx, rows, acc, gsem, ssem, isem = scr[:7]
        if with_edge:
            eavs, esem = scr[7], scr[8]
        cid = lax.axis_index("c")
        sid = lax.axis_index("s")
        ebase = sid * E_CHUNK
        pbase = sid * (E_CHUNK // 2)
        rbase = sid * ACC_SLICE
        ng = jnp.minimum(NGROUP, (E - ebase) // GBE)

        def idx_start(g, x):
            pltpu.async_copy(srcf.at[pl.ds(ebase + g * GBE, GBE)],
                             sidx[x], isem[x])
            pltpu.async_copy(
                dst2.at[pl.ds((ebase + g * GBE) // EDGE_BATCH, GB)],
                didx[x], isem[x])

        def idx_wait(g, x):
            pltpu.make_async_copy(srcf.at[pl.ds(ebase + g * GBE, GBE)],
                                  sidx[x], isem[x]).wait()
            pltpu.make_async_copy(
                dst2.at[pl.ds((ebase + g * GBE) // EDGE_BATCH, GB)],
                didx[x], isem[x]).wait()

        def gather_start(h_ref, r, x):
            pltpu.async_copy(h_ref.at[sidx[x]], rows[r], gsem[r])

        def gather_wait(h_ref, r, x):
            pltpu.make_async_copy(h_ref.at[sidx[x]], rows[r],
                                  gsem[r]).wait()

        def ea_slice(ea_ref, g):
            return ea_ref.at[pl.ds(pbase + g * EDGE_BATCH, EDGE_BATCH)]

        def ea_start(ea_ref, g, e):
            pltpu.async_copy(ea_slice(ea_ref, g), eavs[e], esem[e])

        def ea_wait(ea_ref, g, e):
            pltpu.make_async_copy(ea_slice(ea_ref, g), eavs[e],
                                  esem[e]).wait()

        def relu_group(r, e):
            rb, ebuf = rows[r], eavs[e]
            RU = 2

            def relu_row(i, carry):
                for u in range(RU):
                    row = RU * i + u
                    for sub in range(GB):
                        for c in range(QW // 16):
                            sl = pl.ds(c * 16, 16)
                            el = pl.ds(sub * QW + c * 16, 16)
                            rb[sub * EDGE_BATCH + row, sl] = jnp.maximum(
                                rb[sub * EDGE_BATCH + row, sl]
                                + ebuf[row, el], 0.0)
                return carry
            lax.fori_loop(0, EDGE_BATCH // RU, relu_row, 0)

        def scatter_group(r, x):
            for sub in range(GB):
                pltpu.async_copy(
                    rows[r].at[pl.ds(sub * EDGE_BATCH, EDGE_BATCH)],
                    acc.at[didx[x].at[sub]], ssem[r], add=True)

        def scatter_drain(r, x):
            for sub in range(GB):
                pltpu.make_async_copy(
                    rows[r].at[pl.ds(sub * EDGE_BATCH, EDGE_BATCH)],
                    acc.at[didx[x].at[sub]], ssem[r]).wait()

        def run_quarter(h_ref, ea_ref, agg_ref):
            pltpu.sync_copy(zeros.at[pl.ds(rbase, ACC_SLICE)],
                            acc.at[pl.ds(rbase, ACC_SLICE)])
            plsc.subcore_barrier()

            idx_start(0, 0)
            idx_wait(0, 0)
            gather_start(h_ref, 0, 0)
            if with_edge:
                ea_start(ea_ref, 0, 0)
            idx_start(1, 1)

            def slot(g, s):
                @pl.when(jnp.logical_and(g >= 2, g - 2 < ng))
                def _():
                    scatter_drain((s - 2) % 3, (s - 2) % 4)

                @pl.when(g + 1 < ng)
                def _():
                    idx_wait(g + 1, (s + 1) % 4)
                    gather_start(h_ref, (s + 1) % 3, (s + 1) % 4)
                    if with_edge:
                        ea_start(ea_ref, g + 1, (s + 1) % 2)

                @pl.when(g < ng)
                def _():
                    gather_wait(h_ref, s % 3, s % 4)
                    if with_edge:
                        ea_wait(ea_ref, g, s % 2)
                        relu_group(s % 3, s % 2)
                    scatter_group(s % 3, s % 4)

                @pl.when(g + 2 < ng)
                def _():
                    idx_start(g + 2, (s + 2) % 4)

            def step(i, carry):
                for s in range(UNROLL):
                    slot(UNROLL * i + s, s)
                return carry
            lax.fori_loop(0, SLOTS // UNROLL, step, 0)
            plsc.subcore_barrier()
            pltpu.sync_copy(acc.at[pl.ds(rbase, ACC_SLICE)],
                            agg_ref.at[pl.ds(rbase, ACC_SLICE)])
            plsc.subcore_barrier()

        half = nq // 2

        @pl.when(cid == 0)
        def _():
            for k in range(half):
                run_quarter(hs[k], es[k], aggs[k])

        @pl.when(cid == 1)
        def _():
            for k in range(half, nq):
                run_quarter(hs[k], es[k], aggs[k])

    return pl.kernel(
        body, out_type=out_type, mesh=mesh, scratch_types=scratch,
        compiler_params=pltpu.CompilerParams(use_tc_tiling_on_sc=False))


def _lin1_body(x, W, b, *hq):
    h = _silu(_silu(jnp.dot(x[...], W[...],
                            preferred_element_type=jnp.float32) + b[...]))
    for q in range(NQ):
        hq[q][...] = h[:, q * QW:(q + 1) * QW]


def _lin1(x, W, b):
    return pl.pallas_call(
        _lin1_body,
        grid=(N_BLKS,),
        in_specs=[
            pl.BlockSpec((ROW_BLK, F), lambda i: (i, 0)),
            pl.BlockSpec((F, H), lambda i: (0, 0)),
            pl.BlockSpec((1, H), lambda i: (0, 0)),
        ],
        out_specs=[pl.BlockSpec((ROW_BLK, QW), lambda i: (i, 0))] * NQ,
        out_shape=[jax.ShapeDtypeStruct((N, QW), jnp.float32)] * NQ,
    )(x, W, b)


_EA_BLK = 1280
_EA_BLKS = E // _EA_BLK


def _edge_lin(edge_attr, W, b, qs):
    def ea_body(xe, Wr, br, *eo):
        v = jnp.dot(xe[...], Wr[...],
                    preferred_element_type=jnp.float32) + br[...]
        for o, q in enumerate(qs):
            qc = v[:, q * QW:(q + 1) * QW]
            strips = [jnp.concatenate(
                [qc[256 * s:256 * s + 128, :],
                 qc[256 * s + 128:256 * (s + 1), :]],
                axis=1) for s in range(_EA_BLK // 256)]
            eo[o][...] = jnp.concatenate(strips, axis=0)

    nqs = len(qs)
    return pl.pallas_call(
        ea_body,
        grid=(_EA_BLKS,),
        in_specs=[
            pl.BlockSpec((_EA_BLK, ED), lambda i: (i, 0)),
            pl.BlockSpec((ED, H), lambda i: (0, 0)),
            pl.BlockSpec((1, H), lambda i: (0, 0)),
        ],
        out_specs=[pl.BlockSpec((_EA_BLK // 2, 2 * QW),
                                lambda i: (i, 0))] * nqs,
        out_shape=[jax.ShapeDtypeStruct((EA_REAL, 2 * QW),
                                        jnp.float32)] * nqs,
    )(edge_attr, W, b)


def _conv_body(*refs):
    hq = refs[0:NQ]
    aq = refs[NQ:2 * NQ]
    epsr, W, b = refs[2 * NQ:2 * NQ + 3]
    oq = refs[2 * NQ + 3:]
    acc = None
    for q in range(NQ):
        z = epsr[...] * hq[q][...] + aq[q][...]
        p = jnp.dot(z, W[q * QW:(q + 1) * QW, :],
                    preferred_element_type=jnp.float32)
        acc = p if acc is None else acc + p
    h = _silu(acc + b[...])
    for q in range(NQ):
        oq[q][...] = h[:, q * QW:(q + 1) * QW]


def _conv(hq, aggq, eps_row, W, b):
    return pl.pallas_call(
        _conv_body,
        grid=(N_BLKS,),
        in_specs=(
            [pl.BlockSpec((ROW_BLK, QW), lambda i: (i, 0))] * NQ
            + [pl.BlockSpec((ROW_BLK, QW), lambda i: (i, 0))] * NQ
            + [
                pl.BlockSpec((1, QW), lambda i: (0, 0)),
                pl.BlockSpec((H, H), lambda i: (0, 0)),
                pl.BlockSpec((1, H), lambda i: (0, 0)),
            ]
        ),
        out_specs=[pl.BlockSpec((ROW_BLK, QW), lambda i: (i, 0))] * NQ,
        out_shape=[jax.ShapeDtypeStruct((N, QW), jnp.float32)] * NQ,
    )(*hq, *aggq, eps_row, W, b)


def _pool_body(h0, h1, h2, h3, W1, b1, W2, b2, out, acc):
    i = pl.program_id(0)

    @pl.when(i == 0)
    def _():
        acc[...] = jnp.zeros_like(acc)

    blk = jnp.concatenate([h0[...], h1[...], h2[...], h3[...]], axis=1)
    acc[...] += jnp.sum(blk, axis=0, keepdims=True)

    @pl.when(i == N_BLKS - 1)
    def _():
        g = _silu(acc[...])
        p = _silu(jnp.dot(g, W1[...], preferred_element_type=jnp.float32)
                  + b1[...])
        out[...] = jnp.dot(p, W2[...], preferred_element_type=jnp.float32) \
            + b2[...]


def _pool_mlp(hq, W1, b1, W2p, b2p):
    return pl.pallas_call(
        _pool_body,
        grid=(N_BLKS,),
        in_specs=[pl.BlockSpec((ROW_BLK, QW), lambda i: (i, 0))] * NQ + [
            pl.BlockSpec((H, H // 2), lambda i: (0, 0)),
            pl.BlockSpec((1, H // 2), lambda i: (0, 0)),
            pl.BlockSpec((H // 2, 128), lambda i: (0, 0)),
            pl.BlockSpec((1, 128), lambda i: (0, 0)),
        ],
        out_specs=pl.BlockSpec((1, 128), lambda i: (0, 0)),
        out_shape=jax.ShapeDtypeStruct((1, 128), jnp.float32),
        scratch_shapes=[pltpu.VMEM((1, H), jnp.float32)],
    )(*hq, W1, b1, W2p, b2p)


def kernel(x, edge_index, edge_attr, lin1_W, lin1_b, edgelin_W, edgelin_b,
           eps0, conv0_W, conv0_b, eps1, conv1_W, conv1_b, eps2, conv2_W,
           conv2_b, pred_W1, pred_b1, pred_W2, pred_b2):
    f32 = jnp.float32

    srcf = edge_index[0]
    dst2 = edge_index[1].reshape(E // EDGE_BATCH, EDGE_BATCH)
    zeros = jnp.zeros((N_OUT, QW), f32)

    lin1_b2 = lin1_b.reshape(1, H)
    edgelin_b2 = edgelin_b.reshape(1, H)
    W2p = jnp.zeros((H // 2, 128), f32).at[:, :O].set(pred_W2)
    b2p = jnp.zeros((1, 128), f32).at[0, :O].set(pred_b2)

    hq = _lin1(x, lin1_W, lin1_b2)

    ea01 = _edge_lin(edge_attr, edgelin_W, edgelin_b2, (0, 1))
    aggA = _make_sc_round(True, 2)(hq[0], hq[1], srcf, dst2, zeros, *ea01)
    ea23 = _edge_lin(edge_attr, edgelin_W, edgelin_b2, (2, 3))
    aggB = _make_sc_round(True, 2)(hq[2], hq[3], srcf, dst2, zeros, *ea23)
    aggq = [aggA[0], aggA[1], aggB[0], aggB[1]]
    eps_row = jnp.full((1, QW), 1.0, f32) * (1.0 + eps0)
    hq = _conv(hq, aggq, eps_row, conv0_W, conv0_b.reshape(1, H))

    dummy_ea = ea01
    for epsk, Wk, bk in ((eps1, conv1_W, conv1_b), (eps2, conv2_W, conv2_b)):
        aggq = _make_sc_round(False)(*hq, srcf, dst2, zeros,
                                     *dummy_ea, *dummy_ea)
        eps_row = jnp.full((1, QW), 1.0, f32) * (1.0 + epsk)
        hq = _conv(hq, aggq, eps_row, Wk, bk.reshape(1, H))

    outp = _pool_mlp(hq, pred_W1, pred_b1.reshape(1, H // 2), W2p, b2p)
    return outp[:, :O]
